# baseline (speedup 1.0000x reference)
import functools

import jax
import jax.numpy as jnp
from jax import lax
from jax.experimental import pallas as pl
from jax.experimental.pallas import tpu as pltpu
from jax.experimental.pallas import tpu_sc as plsc

_N = 8192
_D = 768
_E = 8
_BT = 1024
_NB = _N // _BT
_BR = 256
_NBR = _N // _BR
_BM = 256
_P_CAP = _N * 2 + _E * _BM
_NBLK = _P_CAP // _BM
_NW = 32
_TPW = _N // _NW
_CH = 32
_CH2 = 32


def _gate_body(x_ref, wg_ref, bg_ref, m_ref, w_ref, hist_ref):
    xb = x_ref[...]
    logits = jnp.dot(xb, wg_ref[...], preferred_element_type=jnp.float32)
    logits = logits + bg_ref[...]
    mx = jnp.max(logits, axis=-1, keepdims=True)
    ex = jnp.exp(logits - mx)
    p = ex / jnp.sum(ex, axis=-1, keepdims=True)
    i1 = jnp.argmax(logits, axis=-1)
    eids = jax.lax.broadcasted_iota(jnp.int32, logits.shape, 1)
    sel1 = eids == i1[:, None]
    masked = jnp.where(sel1, -jnp.inf, logits)
    i2 = jnp.argmax(masked, axis=-1)
    sel2 = eids == i2[:, None]
    m = jnp.where(sel1, 1, jnp.where(sel2, 2, 0)).astype(jnp.int32)
    m_ref[...] = m
    w_ref[...] = jnp.where(sel1 | sel2, p, 0.0)
    h = (sel1 | sel2).astype(jnp.int32)
    hist_ref[...] = jnp.sum(h, axis=0)[None, None, :]


def _gate(x, W_gate, b_gate2d):
    return pl.pallas_call(
        _gate_body,
        grid=(_NB,),
        in_specs=[
            pl.BlockSpec((_BT, _D), lambda i: (i, 0)),
            pl.BlockSpec((_D, _E), lambda i: (0, 0)),
            pl.BlockSpec((1, _E), lambda i: (0, 0)),
        ],
        out_specs=[
            pl.BlockSpec((_BT, _E), lambda i: (i, 0)),
            pl.BlockSpec((_BT, _E), lambda i: (i, 0)),
            pl.BlockSpec((1, 1, _E), lambda i: (i, 0, 0)),
        ],
        out_shape=[
            jax.ShapeDtypeStruct((_N, _E), jnp.int32),
            jax.ShapeDtypeStruct((_N, _E), jnp.float32),
            jax.ShapeDtypeStruct((_NB, 1, _E), jnp.int32),
        ],
    )(x, W_gate, b_gate2d)


def _route_body(m_ref, w_ref, hist_ref, pos1_ref, pos2_ref, p1_ref, p2_ref,
                blk_ref, carry):
    i = pl.program_id(0)

    @pl.when(i == 0)
    def _():
        carry[...] = jnp.zeros((1, _E), jnp.float32)

    m = m_ref[...]
    w = w_ref[...]
    h1 = (m == 1).astype(jnp.float32)
    h2 = (m == 2).astype(jnp.float32)
    h = h1 + h2

    tot = jnp.sum(hist_ref[...], axis=(0, 1))
    ru = ((tot + _BM - 1) // _BM) * _BM
    fi = jax.lax.broadcasted_iota(jnp.int32, (_E, _E), 0)
    ei = jax.lax.broadcasted_iota(jnp.int32, (_E, _E), 1)
    seg = jnp.sum(jnp.where(fi < ei, ru[:, None], 0), axis=0)
    segf = seg.astype(jnp.float32)[None, :]

    r = jax.lax.broadcasted_iota(jnp.int32, (_BR, _BR), 0)
    c = jax.lax.broadcasted_iota(jnp.int32, (_BR, _BR), 1)
    tri = (r >= c).astype(jnp.float32)
    s = jnp.dot(tri, h, preferred_element_type=jnp.float32)

    a = segf + carry[...] + s - h
    pos1 = jnp.sum(h1 * a, axis=1).astype(jnp.int32)
    pos2 = jnp.sum(h2 * a, axis=1).astype(jnp.int32)
    pos1_ref[...] = pos1[None, None, :]
    pos2_ref[...] = pos2[None, None, :]
    p1_ref[...] = jnp.broadcast_to(jnp.sum(h1 * w, axis=1)[:, None], (_BR, 128))
    p2_ref[...] = jnp.broadcast_to(jnp.sum(h2 * w, axis=1)[:, None], (_BR, 128))
    carry[...] = carry[...] + s[_BR - 1:_BR, :]

    jb = jax.lax.broadcasted_iota(jnp.int32, (_E, _NBLK), 1) * _BM
    cnt = jnp.sum((jb >= seg[:, None]).astype(jnp.int32), axis=0)[None, :]
    blk_ref[...] = jnp.clip(cnt - 1, 0, _E - 1)


def _route(m, w, hist):
    return pl.pallas_call(
        _route_body,
        grid=(_NBR,),
        in_specs=[
            pl.BlockSpec((_BR, _E), lambda i: (i, 0)),
            pl.BlockSpec((_BR, _E), lambda i: (i, 0)),
            pl.BlockSpec((_NB, 1, _E), lambda i: (0, 0, 0)),
        ],
        out_specs=[
            pl.BlockSpec((1, 1, _BR), lambda i: (i, 0, 0)),
            pl.BlockSpec((1, 1, _BR), lambda i: (i, 0, 0)),
            pl.BlockSpec((_BR, 128), lambda i: (i, 0)),
            pl.BlockSpec((_BR, 128), lambda i: (i, 0)),
            pl.BlockSpec((1, _NBLK), lambda i: (0, 0)),
        ],
        out_shape=[
            jax.ShapeDtypeStruct((_NBR, 1, _BR), jnp.int32),
            jax.ShapeDtypeStruct((_NBR, 1, _BR), jnp.int32),
            jax.ShapeDtypeStruct((_N, 128), jnp.float32),
            jax.ShapeDtypeStruct((_N, 128), jnp.float32),
            jax.ShapeDtypeStruct((1, _NBLK), jnp.int32),
        ],
        scratch_shapes=[pltpu.VMEM((1, _E), jnp.float32)],
    )(m, w, hist)


_NCH = _TPW // _CH


def _make_scatter():
    mesh = plsc.VectorSubcoreMesh(core_axis_name="c", subcore_axis_name="s")

    @functools.partial(
        pl.kernel,
        mesh=mesh,
        out_type=[
            jax.ShapeDtypeStruct((_P_CAP, _D), jnp.float32),
            jax.ShapeDtypeStruct((_P_CAP, 128), jnp.float32),
        ],
        scratch_types=[
            pltpu.VMEM((2, _CH, _D), jnp.float32),
            pltpu.VMEM((2, _CH), jnp.int32),
            pltpu.VMEM((2, _CH), jnp.int32),
            pltpu.VMEM((2, _CH, 128), jnp.float32),
            pltpu.VMEM((2, _CH, 128), jnp.float32),
            pltpu.SemaphoreType.DMA,
            pltpu.SemaphoreType.DMA,
            pltpu.SemaphoreType.DMA,
            pltpu.SemaphoreType.DMA,
            pltpu.SemaphoreType.DMA,
            pltpu.SemaphoreType.DMA,
            pltpu.SemaphoreType.DMA,
            pltpu.SemaphoreType.DMA,
        ],
    )
    def scatter_k(x_hbm, pos1_hbm, pos2_hbm, pb1_hbm, pb2_hbm,
                  xs_hbm, ps_hbm,
                  xrows, i1v, i2v, pr1, pr2, *sems):
        wid = lax.axis_index("s") * 2 + lax.axis_index("c")

        def fire(ch, par):
            base = wid * _TPW + ch * _CH
            pltpu.sync_copy(x_hbm.at[pl.ds(base, _CH)], xrows.at[par])
            pltpu.sync_copy(pos1_hbm.at[pl.ds(base, _CH)], i1v.at[par])
            pltpu.sync_copy(pos2_hbm.at[pl.ds(base, _CH)], i2v.at[par])
            pltpu.sync_copy(pb1_hbm.at[pl.ds(base, _CH)], pr1.at[par])
            pltpu.sync_copy(pb2_hbm.at[pl.ds(base, _CH)], pr2.at[par])
            pltpu.async_copy(xrows.at[par], xs_hbm.at[i1v.at[par]],
                             sems[4 * par + 0])
            pltpu.async_copy(xrows.at[par], xs_hbm.at[i2v.at[par]],
                             sems[4 * par + 1])
            pltpu.async_copy(pr1.at[par], ps_hbm.at[i1v.at[par]],
                             sems[4 * par + 2])
            pltpu.async_copy(pr2.at[par], ps_hbm.at[i2v.at[par]],
                             sems[4 * par + 3])

        def drain(ch, par):
            base = wid * _TPW + ch * _CH
            pltpu.make_async_copy(xrows.at[par], xs_hbm.at[i1v.at[par]],
                                  sems[4 * par + 0]).wait()
            pltpu.make_async_copy(xrows.at[par], xs_hbm.at[i2v.at[par]],
                                  sems[4 * par + 1]).wait()
            pltpu.make_async_copy(pr1.at[par], ps_hbm.at[i1v.at[par]],
                                  sems[4 * par + 2]).wait()
            pltpu.make_async_copy(pr2.at[par], ps_hbm.at[i2v.at[par]],
                                  sems[4 * par + 3]).wait()

        fire(0, 0)
        for ch in range(_NCH):
            par = ch & 1
            if ch + 1 < _NCH:
                if ch >= 1:
                    drain(ch - 1, 1 - par)
                fire(ch + 1, 1 - par)
        drain(_NCH - 2, (_NCH - 2) & 1)
        drain(_NCH - 1, (_NCH - 1) & 1)

    return scatter_k


def _mm_body(be_ref, xs_ref, ps_ref, w_ref, b_ref, ys_ref):
    y = jnp.dot(xs_ref[...], w_ref[0], preferred_element_type=jnp.float32)
    y = y + b_ref[0]
    ys_ref[...] = y * ps_ref[:, :1]


def _grouped_mm(blk_e, xs, ps, W_experts, b_experts):
    grid_spec = pltpu.PrefetchScalarGridSpec(
        num_scalar_prefetch=1,
        grid=(_NBLK,),
        in_specs=[
            pl.BlockSpec((_BM, _D), lambda i, be: (i, 0)),
            pl.BlockSpec((_BM, 128), lambda i, be: (i, 0)),
            pl.BlockSpec((1, _D, _D), lambda i, be: (be[i], 0, 0)),
            pl.BlockSpec((1, 1, _D), lambda i, be: (be[i], 0, 0)),
        ],
        out_specs=pl.BlockSpec((_BM, _D), lambda i, be: (i, 0)),
    )
    return pl.pallas_call(
        _mm_body,
        grid_spec=grid_spec,
        out_shape=jax.ShapeDtypeStruct((_P_CAP, _D), jnp.float32),
    )(blk_e, xs, ps, W_experts, b_experts)


def _make_combine():
    mesh = plsc.VectorSubcoreMesh(core_axis_name="c", subcore_axis_name="s")

    @functools.partial(
        pl.kernel,
        mesh=mesh,
        out_type=jax.ShapeDtypeStruct((_N, _D), jnp.float32),
        scratch_types=[
            pltpu.VMEM((2, _CH2), jnp.int32),
            pltpu.VMEM((2, _CH2), jnp.int32),
            pltpu.VMEM((2, _CH2, _D), jnp.float32),
            pltpu.VMEM((2, _CH2, _D), jnp.float32),
            pltpu.SemaphoreType.DMA,
            pltpu.SemaphoreType.DMA,
            pltpu.SemaphoreType.DMA,
            pltpu.SemaphoreType.DMA,
            pltpu.SemaphoreType.DMA,
            pltpu.SemaphoreType.DMA,
        ],
    )
    def combine_k(ys_hbm, pos1_hbm, pos2_hbm, out_hbm,
                  i1v, i2v, r1v, r2v, g1a, g1b, g2a, g2b, woa, wob):
        wid = lax.axis_index("s") * 2 + lax.axis_index("c")
        nch = _TPW // _CH2
        g1 = (g1a, g1b)
        g2 = (g2a, g2b)
        wo = (woa, wob)

        def fire(ch, par):
            base = wid * _TPW + ch * _CH2
            pltpu.sync_copy(pos1_hbm.at[pl.ds(base, _CH2)], i1v.at[par])
            pltpu.sync_copy(pos2_hbm.at[pl.ds(base, _CH2)], i2v.at[par])
            pltpu.async_copy(ys_hbm.at[i1v.at[par]], r1v.at[par], g1[par])
            pltpu.async_copy(ys_hbm.at[i2v.at[par]], r2v.at[par], g2[par])

        fire(0, 0)
        for ch in range(nch):
            par = ch & 1
            if ch + 1 < nch:
                if ch >= 1:
                    obase = wid * _TPW + (ch - 1) * _CH2
                    pltpu.make_async_copy(
                        r1v.at[1 - par],
                        out_hbm.at[pl.ds(obase, _CH2)], wo[1 - par]).wait()
                fire(ch + 1, 1 - par)
            pltpu.make_async_copy(ys_hbm.at[i1v.at[par]], r1v.at[par],
                                  g1[par]).wait()
            pltpu.make_async_copy(ys_hbm.at[i2v.at[par]], r2v.at[par],
                                  g2[par]).wait()

            def body(t, unused):
                for cc in range(_D // 16):
                    sl = pl.ds(cc * 16, 16)
                    r1v[par, t, sl] = r1v[par, t, sl] + r2v[par, t, sl]
                return unused

            lax.fori_loop(0, _CH2, body, 0)
            base = wid * _TPW + ch * _CH2
            pltpu.async_copy(r1v.at[par], out_hbm.at[pl.ds(base, _CH2)],
                             wo[par])
        for ch in (nch - 2, nch - 1):
            par = ch & 1
            base = wid * _TPW + ch * _CH2
            pltpu.make_async_copy(r1v.at[par], out_hbm.at[pl.ds(base, _CH2)],
                                  wo[par]).wait()

    return combine_k


@jax.jit
def _moe_sc(x, W_experts, b_experts, W_gate, b_gate2d):
    m, w, hist = _gate(x, W_gate, b_gate2d)
    pos1, pos2, p1, p2, blk_e = _route(m, w, hist)
    pos1f = pos1.reshape(_N)
    pos2f = pos2.reshape(_N)
    xs, ps = _make_scatter()(x, pos1f, pos2f, p1, p2)
    ys = _grouped_mm(blk_e.reshape(_NBLK), xs, ps, W_experts,
                     b_experts.reshape(_E, 1, _D))
    out = _make_combine()(ys, pos1f, pos2f)
    return out


def kernel(x, W_experts, b_experts, W_gate, b_gate):
    return _moe_sc(x, W_experts, b_experts, W_gate, b_gate.reshape(1, _E))


# speedup vs baseline: 1.0907x; 1.0907x over previous
import functools

import jax
import jax.numpy as jnp
from jax import lax
from jax.experimental import pallas as pl
from jax.experimental.pallas import tpu as pltpu
from jax.experimental.pallas import tpu_sc as plsc

_N = 8192
_D = 768
_E = 8
_BT = 1024
_NB = _N // _BT
_BR = 256
_NBR = _N // _BR
_BM = 256
_P_CAP = _N * 2 + _E * _BM
_NBLK = _P_CAP // _BM
_NW = 32
_TPW = _N // _NW
_CH = 64
_CH2 = 64


def _gate_body(x_ref, wg_ref, bg_ref, m_ref, w_ref, hist_ref):
    xb = x_ref[...]
    logits = jnp.dot(xb, wg_ref[...], preferred_element_type=jnp.float32)
    logits = logits + bg_ref[...]
    mx = jnp.max(logits, axis=-1, keepdims=True)
    ex = jnp.exp(logits - mx)
    p = ex / jnp.sum(ex, axis=-1, keepdims=True)
    i1 = jnp.argmax(logits, axis=-1)
    eids = jax.lax.broadcasted_iota(jnp.int32, logits.shape, 1)
    sel1 = eids == i1[:, None]
    masked = jnp.where(sel1, -jnp.inf, logits)
    i2 = jnp.argmax(masked, axis=-1)
    sel2 = eids == i2[:, None]
    m = jnp.where(sel1, 1, jnp.where(sel2, 2, 0)).astype(jnp.int32)
    m_ref[...] = m
    w_ref[...] = jnp.where(sel1 | sel2, p, 0.0)
    h = (sel1 | sel2).astype(jnp.int32)
    hist_ref[...] = jnp.sum(h, axis=0)[None, None, :]


def _gate(x, W_gate, b_gate2d):
    return pl.pallas_call(
        _gate_body,
        grid=(_NB,),
        in_specs=[
            pl.BlockSpec((_BT, _D), lambda i: (i, 0)),
            pl.BlockSpec((_D, _E), lambda i: (0, 0)),
            pl.BlockSpec((1, _E), lambda i: (0, 0)),
        ],
        out_specs=[
            pl.BlockSpec((_BT, _E), lambda i: (i, 0)),
            pl.BlockSpec((_BT, _E), lambda i: (i, 0)),
            pl.BlockSpec((1, 1, _E), lambda i: (i, 0, 0)),
        ],
        out_shape=[
            jax.ShapeDtypeStruct((_N, _E), jnp.int32),
            jax.ShapeDtypeStruct((_N, _E), jnp.float32),
            jax.ShapeDtypeStruct((_NB, 1, _E), jnp.int32),
        ],
    )(x, W_gate, b_gate2d)


def _route_body(m_ref, w_ref, hist_ref, pos1_ref, pos2_ref, p1_ref, p2_ref,
                blk_ref, carry):
    i = pl.program_id(0)

    @pl.when(i == 0)
    def _():
        carry[...] = jnp.zeros((1, _E), jnp.float32)

    m = m_ref[...]
    w = w_ref[...]
    h1 = (m == 1).astype(jnp.float32)
    h2 = (m == 2).astype(jnp.float32)
    h = h1 + h2

    tot = jnp.sum(hist_ref[...], axis=(0, 1))
    ru = ((tot + _BM - 1) // _BM) * _BM
    fi = jax.lax.broadcasted_iota(jnp.int32, (_E, _E), 0)
    ei = jax.lax.broadcasted_iota(jnp.int32, (_E, _E), 1)
    seg = jnp.sum(jnp.where(fi < ei, ru[:, None], 0), axis=0)
    segf = seg.astype(jnp.float32)[None, :]

    r = jax.lax.broadcasted_iota(jnp.int32, (_BR, _BR), 0)
    c = jax.lax.broadcasted_iota(jnp.int32, (_BR, _BR), 1)
    tri = (r >= c).astype(jnp.float32)
    s = jnp.dot(tri, h, preferred_element_type=jnp.float32)

    a = segf + carry[...] + s - h
    pos1 = jnp.sum(h1 * a, axis=1).astype(jnp.int32)
    pos2 = jnp.sum(h2 * a, axis=1).astype(jnp.int32)
    pos1_ref[...] = pos1[None, None, :]
    pos2_ref[...] = pos2[None, None, :]
    p1_ref[...] = jnp.broadcast_to(jnp.sum(h1 * w, axis=1)[:, None], (_BR, 128))
    p2_ref[...] = jnp.broadcast_to(jnp.sum(h2 * w, axis=1)[:, None], (_BR, 128))
    carry[...] = carry[...] + s[_BR - 1:_BR, :]

    jb = jax.lax.broadcasted_iota(jnp.int32, (_E, _NBLK), 1) * _BM
    cnt = jnp.sum((jb >= seg[:, None]).astype(jnp.int32), axis=0)[None, :]
    blk_ref[...] = jnp.clip(cnt - 1, 0, _E - 1)


def _route(m, w, hist):
    return pl.pallas_call(
        _route_body,
        grid=(_NBR,),
        in_specs=[
            pl.BlockSpec((_BR, _E), lambda i: (i, 0)),
            pl.BlockSpec((_BR, _E), lambda i: (i, 0)),
            pl.BlockSpec((_NB, 1, _E), lambda i: (0, 0, 0)),
        ],
        out_specs=[
            pl.BlockSpec((1, 1, _BR), lambda i: (i, 0, 0)),
            pl.BlockSpec((1, 1, _BR), lambda i: (i, 0, 0)),
            pl.BlockSpec((_BR, 128), lambda i: (i, 0)),
            pl.BlockSpec((_BR, 128), lambda i: (i, 0)),
            pl.BlockSpec((1, _NBLK), lambda i: (0, 0)),
        ],
        out_shape=[
            jax.ShapeDtypeStruct((_NBR, 1, _BR), jnp.int32),
            jax.ShapeDtypeStruct((_NBR, 1, _BR), jnp.int32),
            jax.ShapeDtypeStruct((_N, 128), jnp.float32),
            jax.ShapeDtypeStruct((_N, 128), jnp.float32),
            jax.ShapeDtypeStruct((1, _NBLK), jnp.int32),
        ],
        scratch_shapes=[pltpu.VMEM((1, _E), jnp.float32)],
    )(m, w, hist)


_NCH = _TPW // _CH


def _make_scatter():
    mesh = plsc.VectorSubcoreMesh(core_axis_name="c", subcore_axis_name="s")

    @functools.partial(
        pl.kernel,
        mesh=mesh,
        out_type=[
            jax.ShapeDtypeStruct((_P_CAP, _D), jnp.float32),
            jax.ShapeDtypeStruct((_P_CAP, 128), jnp.float32),
        ],
        scratch_types=[
            pltpu.VMEM((_CH, _D), jnp.float32),
            pltpu.VMEM((_CH,), jnp.int32),
            pltpu.VMEM((_CH,), jnp.int32),
            pltpu.VMEM((_CH, 128), jnp.float32),
            pltpu.VMEM((_CH, 128), jnp.float32),
            pltpu.SemaphoreType.DMA,
            pltpu.SemaphoreType.DMA,
            pltpu.SemaphoreType.DMA,
            pltpu.SemaphoreType.DMA,
        ],
    )
    def scatter_k(x_hbm, pos1_hbm, pos2_hbm, pb1_hbm, pb2_hbm,
                  xs_hbm, ps_hbm,
                  xrows, i1v, i2v, pr1, pr2, s1, s2, s3, s4):
        wid = lax.axis_index("s") * 2 + lax.axis_index("c")
        for ch in range(_NCH):
            base = wid * _TPW + ch * _CH
            pltpu.sync_copy(x_hbm.at[pl.ds(base, _CH)], xrows)
            pltpu.sync_copy(pos1_hbm.at[pl.ds(base, _CH)], i1v)
            pltpu.sync_copy(pos2_hbm.at[pl.ds(base, _CH)], i2v)
            pltpu.sync_copy(pb1_hbm.at[pl.ds(base, _CH)], pr1)
            pltpu.sync_copy(pb2_hbm.at[pl.ds(base, _CH)], pr2)
            c1 = pltpu.async_copy(xrows, xs_hbm.at[i1v], s1)
            c2 = pltpu.async_copy(xrows, xs_hbm.at[i2v], s2)
            c3 = pltpu.async_copy(pr1, ps_hbm.at[i1v], s3)
            c4 = pltpu.async_copy(pr2, ps_hbm.at[i2v], s4)
            c1.wait()
            c2.wait()
            c3.wait()
            c4.wait()

    return scatter_k


def _mm_body(be_ref, xs_ref, ps_ref, w_ref, b_ref, ys_ref):
    y = jnp.dot(xs_ref[...], w_ref[0], preferred_element_type=jnp.float32)
    y = y + b_ref[0]
    ys_ref[...] = y * ps_ref[:, :1]


def _grouped_mm(blk_e, xs, ps, W_experts, b_experts):
    grid_spec = pltpu.PrefetchScalarGridSpec(
        num_scalar_prefetch=1,
        grid=(_NBLK,),
        in_specs=[
            pl.BlockSpec((_BM, _D), lambda i, be: (i, 0)),
            pl.BlockSpec((_BM, 128), lambda i, be: (i, 0)),
            pl.BlockSpec((1, _D, _D), lambda i, be: (be[i], 0, 0)),
            pl.BlockSpec((1, 1, _D), lambda i, be: (be[i], 0, 0)),
        ],
        out_specs=pl.BlockSpec((_BM, _D), lambda i, be: (i, 0)),
    )
    return pl.pallas_call(
        _mm_body,
        grid_spec=grid_spec,
        out_shape=jax.ShapeDtypeStruct((_P_CAP, _D), jnp.float32),
    )(blk_e, xs, ps, W_experts, b_experts)


def _make_combine():
    mesh = plsc.VectorSubcoreMesh(core_axis_name="c", subcore_axis_name="s")

    @functools.partial(
        pl.kernel,
        mesh=mesh,
        out_type=jax.ShapeDtypeStruct((_N, _D), jnp.float32),
        scratch_types=[
            pltpu.VMEM((_CH2,), jnp.int32),
            pltpu.VMEM((_CH2,), jnp.int32),
            pltpu.VMEM((_CH2, _D), jnp.float32),
            pltpu.VMEM((_CH2, _D), jnp.float32),
            pltpu.SemaphoreType.DMA,
            pltpu.SemaphoreType.DMA,
        ],
    )
    def combine_k(ys_hbm, pos1_hbm, pos2_hbm, out_hbm,
                  i1v, i2v, r1v, r2v, s1, s2):
        wid = lax.axis_index("s") * 2 + lax.axis_index("c")
        for ch in range(_TPW // _CH2):
            base = wid * _TPW + ch * _CH2
            pltpu.sync_copy(pos1_hbm.at[pl.ds(base, _CH2)], i1v)
            pltpu.sync_copy(pos2_hbm.at[pl.ds(base, _CH2)], i2v)
            c1 = pltpu.async_copy(ys_hbm.at[i1v], r1v, s1)
            c2 = pltpu.async_copy(ys_hbm.at[i2v], r2v, s2)
            c1.wait()
            c2.wait()

            def body(t, unused):
                for cc in range(_D // 16):
                    sl = pl.ds(cc * 16, 16)
                    r1v[t, sl] = r1v[t, sl] + r2v[t, sl]
                return unused

            lax.fori_loop(0, _CH2, body, 0)
            pltpu.sync_copy(r1v, out_hbm.at[pl.ds(base, _CH2)])

    return combine_k


@jax.jit
def _moe_sc(x, W_experts, b_experts, W_gate, b_gate2d):
    m, w, hist = _gate(x, W_gate, b_gate2d)
    pos1, pos2, p1, p2, blk_e = _route(m, w, hist)
    pos1f = pos1.reshape(_N)
    pos2f = pos2.reshape(_N)
    xs, ps = _make_scatter()(x, pos1f, pos2f, p1, p2)
    ys = _grouped_mm(blk_e.reshape(_NBLK), xs, ps, W_experts,
                     b_experts.reshape(_E, 1, _D))
    out = _make_combine()(ys, pos1f, pos2f)
    return out


def kernel(x, W_experts, b_experts, W_gate, b_gate):
    return _moe_sc(x, W_experts, b_experts, W_gate, b_gate.reshape(1, _E))


# speedup vs baseline: 1.1072x; 1.0151x over previous
import functools

import jax
import jax.numpy as jnp
from jax import lax
from jax.experimental import pallas as pl
from jax.experimental.pallas import tpu as pltpu
from jax.experimental.pallas import tpu_sc as plsc

_N = 8192
_D = 768
_E = 8
_BR = 256
_NBR = _N // _BR
_BM = 256
_P_CAP = _N * 2 + _E * _BM
_NBLK = _P_CAP // _BM
_NW = 32
_TPW = _N // _NW
_CH = 64
_CH2 = 64


_BT = 1024
_NB = _N // _BT


def _gate_body(x_ref, wg_ref, bg_ref, m_ref, w_ref, hist_ref):
    xb = x_ref[...]
    logits = jnp.dot(xb, wg_ref[...], preferred_element_type=jnp.float32)
    logits = logits + bg_ref[...]
    mx = jnp.max(logits, axis=-1, keepdims=True)
    ex = jnp.exp(logits - mx)
    p = ex / jnp.sum(ex, axis=-1, keepdims=True)
    i1 = jnp.argmax(logits, axis=-1)
    eids = jax.lax.broadcasted_iota(jnp.int32, logits.shape, 1)
    sel1 = eids == i1[:, None]
    masked = jnp.where(sel1, -jnp.inf, logits)
    i2 = jnp.argmax(masked, axis=-1)
    sel2 = eids == i2[:, None]
    m_ref[...] = jnp.where(sel1, 1, jnp.where(sel2, 2, 0)).astype(jnp.int32)
    w_ref[...] = jnp.where(sel1 | sel2, p, 0.0)
    hist_ref[...] = jnp.sum((sel1 | sel2).astype(jnp.int32),
                            axis=0)[None, None, :]


def _gate(x, W_gate, b_gate2d):
    return pl.pallas_call(
        _gate_body,
        grid=(_NB,),
        in_specs=[
            pl.BlockSpec((_BT, _D), lambda i: (i, 0)),
            pl.BlockSpec((_D, _E), lambda i: (0, 0)),
            pl.BlockSpec((1, _E), lambda i: (0, 0)),
        ],
        out_specs=[
            pl.BlockSpec((_BT, _E), lambda i: (i, 0)),
            pl.BlockSpec((_BT, _E), lambda i: (i, 0)),
            pl.BlockSpec((1, 1, _E), lambda i: (i, 0, 0)),
        ],
        out_shape=[
            jax.ShapeDtypeStruct((_N, _E), jnp.int32),
            jax.ShapeDtypeStruct((_N, _E), jnp.float32),
            jax.ShapeDtypeStruct((_NB, 1, _E), jnp.int32),
        ],
    )(x, W_gate, b_gate2d)


def _route_body(m_ref, w_ref, hist_ref, pos1_ref, pos2_ref, pb1_ref, pb2_ref,
                blk_ref, carry):
    i = pl.program_id(0)

    @pl.when(i == 0)
    def _():
        carry[...] = jnp.zeros((1, _E), jnp.float32)

    m = m_ref[...]
    w = w_ref[...]
    h1 = (m == 1).astype(jnp.float32)
    h2 = (m == 2).astype(jnp.float32)
    h = h1 + h2

    tot = jnp.sum(hist_ref[...], axis=(0, 1))
    ru = ((tot + _BM - 1) // _BM) * _BM
    fi = jax.lax.broadcasted_iota(jnp.int32, (_E, _E), 0)
    ei = jax.lax.broadcasted_iota(jnp.int32, (_E, _E), 1)
    seg = jnp.sum(jnp.where(fi < ei, ru[:, None], 0), axis=0)
    segf = seg.astype(jnp.float32)[None, :]

    r = jax.lax.broadcasted_iota(jnp.int32, (_BR, _BR), 0)
    c = jax.lax.broadcasted_iota(jnp.int32, (_BR, _BR), 1)
    tri = (r >= c).astype(jnp.float32)
    s = jnp.dot(tri, h, preferred_element_type=jnp.float32)

    a = segf + carry[...] + s - h
    pos1 = jnp.sum(h1 * a, axis=1).astype(jnp.int32)
    pos2 = jnp.sum(h2 * a, axis=1).astype(jnp.int32)
    pos1_ref[...] = pos1[None, None, :]
    pos2_ref[...] = pos2[None, None, :]
    pb1_ref[...] = jnp.broadcast_to(
        jnp.sum(h1 * w, axis=1)[:, None], (_BR, 16))
    pb2_ref[...] = jnp.broadcast_to(
        jnp.sum(h2 * w, axis=1)[:, None], (_BR, 16))
    carry[...] = carry[...] + s[_BR - 1:_BR, :]

    jb = jax.lax.broadcasted_iota(jnp.int32, (_E, _NBLK), 1) * _BM
    cnt = jnp.sum((jb >= seg[:, None]).astype(jnp.int32), axis=0)[None, :]
    blk_ref[...] = jnp.clip(cnt - 1, 0, _E - 1)


def _route(m, w, hist):
    return pl.pallas_call(
        _route_body,
        grid=(_NBR,),
        in_specs=[
            pl.BlockSpec((_BR, _E), lambda i: (i, 0)),
            pl.BlockSpec((_BR, _E), lambda i: (i, 0)),
            pl.BlockSpec((_NB, 1, _E), lambda i: (0, 0, 0)),
        ],
        out_specs=[
            pl.BlockSpec((1, 1, _BR), lambda i: (i, 0, 0)),
            pl.BlockSpec((1, 1, _BR), lambda i: (i, 0, 0)),
            pl.BlockSpec((_BR, 16), lambda i: (i, 0)),
            pl.BlockSpec((_BR, 16), lambda i: (i, 0)),
            pl.BlockSpec((1, _NBLK), lambda i: (0, 0)),
        ],
        out_shape=[
            jax.ShapeDtypeStruct((_NBR, 1, _BR), jnp.int32),
            jax.ShapeDtypeStruct((_NBR, 1, _BR), jnp.int32),
            jax.ShapeDtypeStruct((_N, 16), jnp.float32),
            jax.ShapeDtypeStruct((_N, 16), jnp.float32),
            jax.ShapeDtypeStruct((1, _NBLK), jnp.int32),
        ],
        scratch_shapes=[pltpu.VMEM((1, _E), jnp.float32)],
    )(m, w, hist)


def _gate_route(x, W_gate, b_gate2d):
    m, w, hist = _gate(x, W_gate, b_gate2d)
    return _route(m, w, hist)


_NCH = _TPW // _CH


def _make_scatter():
    mesh = plsc.VectorSubcoreMesh(core_axis_name="c", subcore_axis_name="s")

    @functools.partial(
        pl.kernel,
        mesh=mesh,
        out_type=jax.ShapeDtypeStruct((_P_CAP, _D), jnp.float32),
        scratch_types=[
            pltpu.VMEM((_CH, _D), jnp.float32),
            pltpu.VMEM((_CH,), jnp.int32),
            pltpu.VMEM((_CH,), jnp.int32),
            pltpu.SemaphoreType.DMA,
            pltpu.SemaphoreType.DMA,
        ],
    )
    def scatter_k(x_hbm, pos1_hbm, pos2_hbm, xs_hbm,
                  xrows, i1v, i2v, s1, s2):
        wid = lax.axis_index("s") * 2 + lax.axis_index("c")
        for ch in range(_NCH):
            base = wid * _TPW + ch * _CH
            off = ch * _CH
            pltpu.sync_copy(x_hbm.at[pl.ds(base, _CH)], xrows)
            pltpu.sync_copy(pos1_hbm.at[wid, 0, pl.ds(off, _CH)], i1v)
            pltpu.sync_copy(pos2_hbm.at[wid, 0, pl.ds(off, _CH)], i2v)
            c1 = pltpu.async_copy(xrows, xs_hbm.at[i1v], s1)
            c2 = pltpu.async_copy(xrows, xs_hbm.at[i2v], s2)
            c1.wait()
            c2.wait()

    return scatter_k


def _mm_body(be_ref, xs_ref, w_ref, b_ref, ys_ref):
    y = jnp.dot(xs_ref[...], w_ref[0], preferred_element_type=jnp.float32)
    ys_ref[...] = y + b_ref[0]


def _grouped_mm(blk_e, xs, W_experts, b_experts):
    grid_spec = pltpu.PrefetchScalarGridSpec(
        num_scalar_prefetch=1,
        grid=(_NBLK,),
        in_specs=[
            pl.BlockSpec((_BM, _D), lambda i, be: (i, 0)),
            pl.BlockSpec((1, _D, _D), lambda i, be: (be[0, i], 0, 0)),
            pl.BlockSpec((1, 1, _D), lambda i, be: (be[0, i], 0, 0)),
        ],
        out_specs=pl.BlockSpec((_BM, _D), lambda i, be: (i, 0)),
    )
    return pl.pallas_call(
        _mm_body,
        grid_spec=grid_spec,
        out_shape=jax.ShapeDtypeStruct((_P_CAP, _D), jnp.float32),
    )(blk_e, xs, W_experts, b_experts)


def _make_combine():
    mesh = plsc.VectorSubcoreMesh(core_axis_name="c", subcore_axis_name="s")

    @functools.partial(
        pl.kernel,
        mesh=mesh,
        out_type=jax.ShapeDtypeStruct((_N, _D), jnp.float32),
        scratch_types=[
            pltpu.VMEM((_CH2,), jnp.int32),
            pltpu.VMEM((_CH2,), jnp.int32),
            pltpu.VMEM((_CH2, _D), jnp.float32),
            pltpu.VMEM((_CH2, _D), jnp.float32),
            pltpu.VMEM((_CH2, 16), jnp.float32),
            pltpu.VMEM((_CH2, 16), jnp.float32),
            pltpu.SemaphoreType.DMA,
            pltpu.SemaphoreType.DMA,
        ],
    )
    def combine_k(ys_hbm, pos1_hbm, pos2_hbm, pb1_hbm, pb2_hbm, out_hbm,
                  i1v, i2v, r1v, r2v, p1v, p2v, s1, s2):
        wid = lax.axis_index("s") * 2 + lax.axis_index("c")
        for ch in range(_TPW // _CH2):
            base = wid * _TPW + ch * _CH2
            off = ch * _CH2
            pltpu.sync_copy(pos1_hbm.at[wid, 0, pl.ds(off, _CH2)], i1v)
            pltpu.sync_copy(pos2_hbm.at[wid, 0, pl.ds(off, _CH2)], i2v)
            pltpu.sync_copy(pb1_hbm.at[pl.ds(base, _CH2)], p1v)
            pltpu.sync_copy(pb2_hbm.at[pl.ds(base, _CH2)], p2v)
            c1 = pltpu.async_copy(ys_hbm.at[i1v], r1v, s1)
            c2 = pltpu.async_copy(ys_hbm.at[i2v], r2v, s2)
            c1.wait()
            c2.wait()

            def body(t, unused):
                pa = p1v[t, pl.ds(0, 16)]
                pb = p2v[t, pl.ds(0, 16)]
                for cc in range(_D // 16):
                    sl = pl.ds(cc * 16, 16)
                    r1v[t, sl] = r1v[t, sl] * pa + r2v[t, sl] * pb
                return unused

            lax.fori_loop(0, _CH2, body, 0)
            pltpu.sync_copy(r1v, out_hbm.at[pl.ds(base, _CH2)])

    return combine_k


@jax.jit
def _moe_sc(x, W_experts, b_experts, W_gate, b_gate2d):
    pos1, pos2, pb1, pb2, blk_e = _gate_route(x, W_gate, b_gate2d)
    xs = _make_scatter()(x, pos1, pos2)
    ys = _grouped_mm(blk_e, xs, W_experts, b_experts.reshape(_E, 1, _D))
    out = _make_combine()(ys, pos1, pos2, pb1, pb2)
    return out


def kernel(x, W_experts, b_experts, W_gate, b_gate):
    return _moe_sc(x, W_experts, b_experts, W_gate, b_gate.reshape(1, _E))
